# Initial kernel scaffold; baseline (speedup 1.0000x reference)
import jax
import jax.numpy as jnp
from jax.experimental import pallas as pl


def kernel(input_feature, edge_index, adj_values, W, b):
    raise NotImplementedError("write your pallas kernel here")


# speedup vs baseline: 3.4502x; 3.4502x over previous
import functools

import jax
import jax.numpy as jnp
from jax import lax
from jax.experimental import pallas as pl
from jax.experimental.pallas import tpu as pltpu
from jax.experimental.pallas import tpu_sc as plsc

N = 10000
D = 128
E = 320000

NC = 2
NS = 16
NW = NC * NS
K = 128
EPW = 10112
E_PAD = EPW * NW
NCHUNK = EPW // K
ZR = 80
NZC = N // ZR


def _mm_body(x_ref, w_ref, o_ref):
    o_ref[...] = jnp.dot(x_ref[...], w_ref[...],
                         preferred_element_type=jnp.float32)


_matmul = pl.pallas_call(
    _mm_body,
    grid=(25,),
    in_specs=[
        pl.BlockSpec((400, D), lambda i: (i, 0)),
        pl.BlockSpec((D, D), lambda i: (0, 0)),
    ],
    out_specs=pl.BlockSpec((400, D), lambda i: (i, 0)),
    out_shape=jax.ShapeDtypeStruct((N, D), jnp.float32),
)


def _comb_body(p0_ref, p1_ref, b_ref, o_ref):
    o_ref[...] = p0_ref[...] + p1_ref[...] + b_ref[...]


_combine = pl.pallas_call(
    _comb_body,
    grid=(25,),
    in_specs=[
        pl.BlockSpec((400, D), lambda i: (i, 0)),
        pl.BlockSpec((400, D), lambda i: (i + 25, 0)),
        pl.BlockSpec((1, D), lambda i: (0, 0)),
    ],
    out_specs=pl.BlockSpec((400, D), lambda i: (i, 0)),
    out_shape=jax.ShapeDtypeStruct((N, D), jnp.float32),
)

_sc_mesh = plsc.VectorSubcoreMesh(
    core_axis_name="c", subcore_axis_name="s", num_cores=NC, num_subcores=NS)


@functools.partial(
    pl.kernel,
    out_type=jax.ShapeDtypeStruct((NC * N, D), jnp.float32),
    mesh=_sc_mesh,
    scratch_types=[
        pltpu.VMEM_SHARED((N, D), jnp.float32),
        pltpu.VMEM((K,), jnp.int32),
        pltpu.VMEM((K,), jnp.int32),
        pltpu.VMEM((K,), jnp.float32),
        pltpu.VMEM((K, D), jnp.float32),
        pltpu.SemaphoreType.DMA,
    ],
)
def _sc_aggregate(src_hbm, dst_hbm, val_hbm, sup_hbm, out_hbm,
                  acc, sidx, didx, vv, rows, sem):
    c = lax.axis_index("c")
    s = lax.axis_index("s")
    wid = c * NS + s

    zero16 = jnp.zeros((16,), jnp.float32)

    def _zrow(r, carry):
        for c8 in range(D // 16):
            rows[r, pl.ds(c8 * 16, 16)] = zero16
        return carry

    lax.fori_loop(0, K, _zrow, 0)
    for i in range(8):
        cid = s + i * NS
        @pl.when(cid < NZC)
        def _():
            pltpu.sync_copy(rows.at[pl.ds(0, ZR)],
                            acc.at[pl.ds(cid * ZR, ZR)])
    plsc.subcore_barrier()

    ebase = wid * EPW

    def _chunk(j, carry):
        base = ebase + j * K
        pltpu.sync_copy(src_hbm.at[pl.ds(base, K)], sidx)
        pltpu.sync_copy(dst_hbm.at[pl.ds(base, K)], didx)
        pltpu.sync_copy(val_hbm.at[pl.ds(base, K)], vv)
        pltpu.async_copy(sup_hbm.at[sidx], rows, sem).wait()

        def _mul(g, inner):
            v16 = vv[pl.ds(g * 16, 16)]
            for i in range(16):
                vb = jnp.full((16,), v16[i], jnp.float32)
                r = g * 16 + i
                for c8 in range(D // 16):
                    sl = pl.ds(c8 * 16, 16)
                    rows[r, sl] = rows[r, sl] * vb
            return inner

        lax.fori_loop(0, K // 16, _mul, 0)

        pltpu.sync_copy(rows, acc.at[didx], add=True)
        return carry

    lax.fori_loop(0, NCHUNK, _chunk, 0)

    plsc.subcore_barrier()
    for i in range(8):
        cid = s + i * NS
        @pl.when(cid < NZC)
        def _():
            pltpu.sync_copy(acc.at[pl.ds(cid * ZR, ZR)],
                            out_hbm.at[pl.ds(c * N + cid * ZR, ZR)])


def kernel(input_feature, edge_index, adj_values, W, b):
    support = _matmul(input_feature, W)

    pad = E_PAD - E
    src = jnp.concatenate([edge_index[0], jnp.zeros((pad,), jnp.int32)])
    dst = jnp.concatenate([edge_index[1], jnp.zeros((pad,), jnp.int32)])
    vals = jnp.concatenate([adj_values, jnp.zeros((pad,), jnp.float32)])

    parts = _sc_aggregate(src, dst, vals, support)
    return _combine(parts, parts, b.reshape(1, D))
